# baseline (speedup 1.0000x reference)
import jax
import jax.numpy as jnp
from jax import lax
from jax.experimental import pallas as pl
from jax.experimental.pallas import tpu as pltpu
from jax.experimental.pallas import tpu_sc as plsc

N = 10000
E = 320000
NC = 2
NS = 16
CH = 128
CR = 160
NBUF = 4
AHEAD = 2
EROWS = NS * CR
EPAD = EROWS * CH
NP = 10240
DUMP = NP - 2
ZR = NP // NS
ZC = 32
CW = 16
ROWBLK = 1000


def _sc_pass(ph, src2d, dst2d, with_count):
    d2 = ph.shape[2]
    mesh = plsc.VectorSubcoreMesh(
        core_axis_name="c", subcore_axis_name="s", num_cores=NC,
        num_subcores=NS)

    out_type = [jax.ShapeDtypeStruct((NC, NP, d2), jnp.float32)]
    if with_count:
        out_type.append(jax.ShapeDtypeStruct((NP, CW), jnp.float32))

    scratch = dict(
        src_buf=pltpu.VMEM((CR, CH), jnp.int32),
        dst_buf=pltpu.VMEM((CR, CH), jnp.int32),
        zbuf=pltpu.VMEM((ZC, d2), jnp.float32),
        acc=pltpu.VMEM_SHARED((NP, d2), jnp.float32),
    )
    for b in range(NBUF):
        scratch[f"rows{b}"] = pltpu.VMEM((CH, d2), jnp.float32)
        scratch[f"semg{b}"] = pltpu.SemaphoreType.DMA
        scratch[f"sems{b}"] = pltpu.SemaphoreType.DMA
    if with_count:
        scratch.update(
            ones_v=pltpu.VMEM((CH, CW), jnp.float32),
            czbuf=pltpu.VMEM((ZC, CW), jnp.float32),
            cacc=pltpu.VMEM_SHARED((NP, CW), jnp.float32),
        )

    def body(ph_hbm, src_hbm, dst_hbm, *outs, **scr):
        src_buf, dst_buf = scr["src_buf"], scr["dst_buf"]
        rows = [scr[f"rows{b}"] for b in range(NBUF)]
        semg = [scr[f"semg{b}"] for b in range(NBUF)]
        sems = [scr[f"sems{b}"] for b in range(NBUF)]
        zbuf, acc = scr["zbuf"], scr["acc"]
        out_hbm = outs[0]
        cid = lax.axis_index("c")
        sid = lax.axis_index("s")
        is_c0 = cid == 0
        tbl = ph_hbm.at[cid]

        pltpu.sync_copy(src_hbm.at[pl.ds(sid * CR, CR)], src_buf)
        pltpu.sync_copy(dst_hbm.at[pl.ds(sid * CR, CR)], dst_buf)

        zeros16 = jnp.zeros((16,), jnp.float32)
        ones16 = jnp.ones((16,), jnp.float32)
        dl = d2 // 16

        def zfill(i, _):
            zbuf[i // dl, pl.ds((i % dl) * 16, 16)] = zeros16
            return 0
        lax.fori_loop(0, ZC * dl, zfill, 0)

        def zcopy(j, _):
            pltpu.sync_copy(zbuf, acc.at[pl.ds(sid * ZR + j * ZC, ZC)])
            return 0
        lax.fori_loop(0, ZR // ZC, zcopy, 0)

        if with_count:
            czbuf, cacc, ones_v = scr["czbuf"], scr["cacc"], scr["ones_v"]

            def czfill(i, _):
                czbuf[i, pl.ds(0, 16)] = zeros16
                return 0
            lax.fori_loop(0, ZC, czfill, 0)

            def ofill(i, _):
                ones_v[i, pl.ds(0, 16)] = ones16
                return 0
            lax.fori_loop(0, CH, ofill, 0)

            def czcopy(j, _):
                pltpu.sync_copy(czbuf, cacc.at[pl.ds(sid * ZR + j * ZC, ZC)])
                return 0
            lax.fori_loop(0, ZR // ZC, czcopy, 0)

        for b in range(1 if with_count else AHEAD):
            pltpu.async_copy(tbl.at[src_buf.at[b]], rows[b], semg[b])
        plsc.subcore_barrier()

        if with_count:
            def cstep(i, _):
                for b in range(2):
                    c = i * 2 + b
                    pltpu.make_async_copy(tbl.at[src_buf.at[0]], rows[b],
                                          semg[b]).wait()

                    @pl.when(c + 1 < CR)
                    def _():
                        pltpu.async_copy(tbl.at[src_buf.at[c + 1]],
                                         rows[1 - b], semg[1 - b])
                    pltpu.sync_copy(rows[b], acc.at[dst_buf.at[c]],
                                    add=True)

                    @pl.when(is_c0)
                    def _():
                        pltpu.sync_copy(scr["ones_v"],
                                        scr["cacc"].at[dst_buf.at[c]],
                                        add=True)
                return 0
            lax.fori_loop(0, CR // 2, cstep, 0)
        else:
            def step(i, _):
                for b in range(NBUF):
                    c = i * NBUF + b
                    pltpu.make_async_copy(tbl.at[src_buf.at[0]], rows[b],
                                          semg[b]).wait()
                    pltpu.async_copy(rows[b], acc.at[dst_buf.at[c]],
                                     sems[b], add=True)
                    c2 = c + AHEAD
                    b2 = (b + AHEAD) % NBUF

                    @pl.when(c2 < CR)
                    def _():
                        @pl.when(c2 >= NBUF)
                        def _():
                            pltpu.make_async_copy(
                                rows[b2], acc.at[dst_buf.at[0]],
                                sems[b2]).wait()
                        pltpu.async_copy(tbl.at[src_buf.at[c2]], rows[b2],
                                         semg[b2])
                return 0
            lax.fori_loop(0, CR // NBUF, step, 0)

            for b in range(NBUF):
                pltpu.make_async_copy(rows[b], acc.at[dst_buf.at[0]],
                                      sems[b]).wait()

        plsc.subcore_barrier()

        pltpu.sync_copy(acc.at[pl.ds(sid * ZR, ZR)],
                        out_hbm.at[cid, pl.ds(sid * ZR, ZR)])
        if with_count:
            @pl.when(is_c0)
            def _():
                pltpu.sync_copy(scr["cacc"].at[pl.ds(sid * ZR, ZR)],
                                outs[1].at[pl.ds(sid * ZR, ZR)])

    fn = pl.kernel(body, out_type=tuple(out_type), mesh=mesh,
                   scratch_types=scratch,
                   compiler_params=pltpu.CompilerParams(
                       use_tc_tiling_on_sc=False))
    return fn(ph, src2d, dst2d)


def _tc_pre(x, wt):
    n, k = x.shape
    m = wt.shape[1]
    m2 = m // 2

    def body(x_ref, w_ref, o_ref):
        p = jnp.dot(x_ref[...], w_ref[...],
                    preferred_element_type=jnp.float32)
        o_ref[0] = p[:, :m2]
        o_ref[1] = p[:, m2:]

    return pl.pallas_call(
        body,
        grid=(n // ROWBLK,),
        in_specs=[
            pl.BlockSpec((ROWBLK, k), lambda i: (i, 0)),
            pl.BlockSpec((k, m), lambda i: (0, 0)),
        ],
        out_specs=pl.BlockSpec((NC, ROWBLK, m2), lambda i: (0, i, 0)),
        out_shape=jax.ShapeDtypeStruct((NC, n, m2), jnp.float32),
    )(x, wt)


def _tc_post(agg, cnt, h_in, wrt, bl, wlnt):
    n, d_in = h_in.shape
    d = wrt.shape[1]
    d2 = d // 2
    dn = wlnt.shape[1]
    dn2 = dn // 2

    def body(a_ref, c_ref, h_ref, wr_ref, bl_ref, wl_ref, ho_ref, po_ref):
        c = c_ref[:, 0:1]
        inv = 1.0 / jnp.maximum(c, 1.0)
        root = jnp.dot(h_ref[...], wr_ref[...],
                       preferred_element_type=jnp.float32)
        a = jnp.concatenate([a_ref[0], a_ref[1]], axis=1)
        h = jnp.maximum(a * inv + bl_ref[...] + root, 0.0)
        ho_ref[...] = h
        p = jnp.dot(h, wl_ref[...], preferred_element_type=jnp.float32)
        po_ref[0] = p[:, :dn2]
        po_ref[1] = p[:, dn2:]

    return pl.pallas_call(
        body,
        grid=(n // ROWBLK,),
        in_specs=[
            pl.BlockSpec((NC, ROWBLK, d2), lambda i: (0, i, 0)),
            pl.BlockSpec((ROWBLK, CW), lambda i: (i, 0)),
            pl.BlockSpec((ROWBLK, d_in), lambda i: (i, 0)),
            pl.BlockSpec((d_in, d), lambda i: (0, 0)),
            pl.BlockSpec((1, d), lambda i: (0, 0)),
            pl.BlockSpec((d, dn), lambda i: (0, 0)),
        ],
        out_specs=[
            pl.BlockSpec((ROWBLK, d), lambda i: (i, 0)),
            pl.BlockSpec((NC, ROWBLK, dn2), lambda i: (0, i, 0)),
        ],
        out_shape=[
            jax.ShapeDtypeStruct((n, d), jnp.float32),
            jax.ShapeDtypeStruct((NC, n, dn2), jnp.float32),
        ],
    )(agg, cnt, h_in, wrt, bl, wlnt)


def _tc_final(agg, cnt, h_in, wrt, bl, wct, bc):
    n, d_in = h_in.shape
    d = wrt.shape[1]
    d2 = d // 2
    m = wct.shape[1]

    def body(a_ref, c_ref, h_ref, wr_ref, bl_ref, wc_ref, bc_ref, o_ref):
        c = c_ref[:, 0:1]
        inv = 1.0 / jnp.maximum(c, 1.0)
        root = jnp.dot(h_ref[...], wr_ref[...],
                       preferred_element_type=jnp.float32)
        a = jnp.concatenate([a_ref[0], a_ref[1]], axis=1)
        h = jnp.maximum(a * inv + bl_ref[...] + root, 0.0)
        o_ref[...] = jnp.dot(h, wc_ref[...],
                             preferred_element_type=jnp.float32) + bc_ref[...]

    return pl.pallas_call(
        body,
        grid=(n // ROWBLK,),
        in_specs=[
            pl.BlockSpec((NC, ROWBLK, d2), lambda i: (0, i, 0)),
            pl.BlockSpec((ROWBLK, CW), lambda i: (i, 0)),
            pl.BlockSpec((ROWBLK, d_in), lambda i: (i, 0)),
            pl.BlockSpec((d_in, d), lambda i: (0, 0)),
            pl.BlockSpec((1, d), lambda i: (0, 0)),
            pl.BlockSpec((d, m), lambda i: (0, 0)),
            pl.BlockSpec((1, m), lambda i: (0, 0)),
        ],
        out_specs=pl.BlockSpec((ROWBLK, m), lambda i: (i, 0)),
        out_shape=jax.ShapeDtypeStruct((n, m), jnp.float32),
    )(agg, cnt, h_in, wrt, bl, wct, bc)


@jax.jit
def kernel(x, edge_index, Wl1, bl1, Wr1, Wl2, bl2, Wr2, Wl3, bl3, Wr3,
           Wc, bc):
    pad_s = jnp.zeros((EPAD - E,), jnp.int32)
    pad_d = jnp.full((EPAD - E,), DUMP, jnp.int32)
    src2d = jnp.concatenate([edge_index[0], pad_s]).reshape(EROWS, CH)
    dst2d = jnp.concatenate([edge_index[1], pad_d]).reshape(EROWS, CH)

    p1 = _tc_pre(x, Wl1.T)
    agg1, cnt = _sc_pass(p1, src2d, dst2d, True)
    h1, p2 = _tc_post(agg1, cnt, x, Wr1.T, bl1.reshape(1, -1), Wl2.T)
    agg2, = _sc_pass(p2, src2d, dst2d, False)
    h2, p3 = _tc_post(agg2, cnt, h1, Wr2.T, bl2.reshape(1, -1), Wl3.T)
    agg3, = _sc_pass(p3, src2d, dst2d, False)
    return _tc_final(agg3, cnt, h2, Wr3.T, bl3.reshape(1, -1), Wc.T,
                     bc.reshape(1, -1))


# speedup vs baseline: 1.1030x; 1.1030x over previous
import jax
import jax.numpy as jnp
from jax import lax
from jax.experimental import pallas as pl
from jax.experimental.pallas import tpu as pltpu
from jax.experimental.pallas import tpu_sc as plsc

N = 10000
E = 320000
NC = 2
NS = 16
CH = 128
CR = 160
NBUF = 4
AHEAD = 2
EROWS = NS * CR
EPAD = EROWS * CH
NP = 10240
DUMP = NP - 2
ZR = NP // NS
ZC = 32
CW = 16
ROWBLK = 1000


def _sc_pass(ph, src2d, dst2d, with_count):
    d2 = ph.shape[2]
    nbuf = NBUF if with_count else 8
    ahead = AHEAD if with_count else 4
    mesh = plsc.VectorSubcoreMesh(
        core_axis_name="c", subcore_axis_name="s", num_cores=NC,
        num_subcores=NS)

    out_type = [jax.ShapeDtypeStruct((NC, NP, d2), jnp.float32)]
    if with_count:
        out_type.append(jax.ShapeDtypeStruct((NP, CW), jnp.float32))

    scratch = dict(
        src_buf=pltpu.VMEM((CR, CH), jnp.int32),
        dst_buf=pltpu.VMEM((CR, CH), jnp.int32),
        zbuf=pltpu.VMEM((ZC, d2), jnp.float32),
        acc=pltpu.VMEM_SHARED((NP, d2), jnp.float32),
    )
    for b in range(nbuf):
        scratch[f"rows{b}"] = pltpu.VMEM((CH, d2), jnp.float32)
        scratch[f"semg{b}"] = pltpu.SemaphoreType.DMA
        scratch[f"sems{b}"] = pltpu.SemaphoreType.DMA
    if with_count:
        scratch.update(
            ones_v=pltpu.VMEM((CH, CW), jnp.float32),
            czbuf=pltpu.VMEM((ZC, CW), jnp.float32),
            cacc=pltpu.VMEM_SHARED((NP, CW), jnp.float32),
        )

    def body(ph_hbm, src_hbm, dst_hbm, *outs, **scr):
        src_buf, dst_buf = scr["src_buf"], scr["dst_buf"]
        rows = [scr[f"rows{b}"] for b in range(nbuf)]
        semg = [scr[f"semg{b}"] for b in range(nbuf)]
        sems = [scr[f"sems{b}"] for b in range(nbuf)]
        zbuf, acc = scr["zbuf"], scr["acc"]
        out_hbm = outs[0]
        cid = lax.axis_index("c")
        sid = lax.axis_index("s")
        is_c0 = cid == 0
        tbl = ph_hbm.at[cid]

        pltpu.sync_copy(src_hbm.at[pl.ds(sid * CR, CR)], src_buf)
        pltpu.sync_copy(dst_hbm.at[pl.ds(sid * CR, CR)], dst_buf)

        zeros16 = jnp.zeros((16,), jnp.float32)
        ones16 = jnp.ones((16,), jnp.float32)
        dl = d2 // 16

        def zfill(i, _):
            zbuf[i // dl, pl.ds((i % dl) * 16, 16)] = zeros16
            return 0
        lax.fori_loop(0, ZC * dl, zfill, 0)

        def zcopy(j, _):
            pltpu.sync_copy(zbuf, acc.at[pl.ds(sid * ZR + j * ZC, ZC)])
            return 0
        lax.fori_loop(0, ZR // ZC, zcopy, 0)

        if with_count:
            czbuf, cacc, ones_v = scr["czbuf"], scr["cacc"], scr["ones_v"]

            def czfill(i, _):
                czbuf[i, pl.ds(0, 16)] = zeros16
                return 0
            lax.fori_loop(0, ZC, czfill, 0)

            def ofill(i, _):
                ones_v[i, pl.ds(0, 16)] = ones16
                return 0
            lax.fori_loop(0, CH, ofill, 0)

            def czcopy(j, _):
                pltpu.sync_copy(czbuf, cacc.at[pl.ds(sid * ZR + j * ZC, ZC)])
                return 0
            lax.fori_loop(0, ZR // ZC, czcopy, 0)

        for b in range(ahead):
            pltpu.async_copy(tbl.at[src_buf.at[b]], rows[b], semg[b])
        plsc.subcore_barrier()

        def step(i, _):
            for b in range(nbuf):
                c = i * nbuf + b
                pltpu.make_async_copy(tbl.at[src_buf.at[0]], rows[b],
                                      semg[b]).wait()
                pltpu.async_copy(rows[b], acc.at[dst_buf.at[c]], sems[b],
                                 add=True)
                if with_count:
                    @pl.when(is_c0)
                    def _():
                        pltpu.sync_copy(scr["ones_v"],
                                        scr["cacc"].at[dst_buf.at[c]],
                                        add=True)
                c2 = c + ahead
                b2 = (b + ahead) % nbuf

                @pl.when(c2 < CR)
                def _():
                    @pl.when(c2 >= nbuf)
                    def _():
                        pltpu.make_async_copy(
                            rows[b2], acc.at[dst_buf.at[0]],
                            sems[b2]).wait()
                    pltpu.async_copy(tbl.at[src_buf.at[c2]], rows[b2],
                                     semg[b2])
            return 0
        lax.fori_loop(0, CR // nbuf, step, 0)

        for b in range(nbuf):
            pltpu.make_async_copy(rows[b], acc.at[dst_buf.at[0]],
                                  sems[b]).wait()

        plsc.subcore_barrier()

        pltpu.sync_copy(acc.at[pl.ds(sid * ZR, ZR)],
                        out_hbm.at[cid, pl.ds(sid * ZR, ZR)])
        if with_count:
            @pl.when(is_c0)
            def _():
                pltpu.sync_copy(scr["cacc"].at[pl.ds(sid * ZR, ZR)],
                                outs[1].at[pl.ds(sid * ZR, ZR)])

    fn = pl.kernel(body, out_type=tuple(out_type), mesh=mesh,
                   scratch_types=scratch,
                   compiler_params=pltpu.CompilerParams(
                       use_tc_tiling_on_sc=False))
    return fn(ph, src2d, dst2d)


def _tc_pre(x, wt):
    n, k = x.shape
    m = wt.shape[1]
    m2 = m // 2

    def body(x_ref, w_ref, o_ref):
        p = jnp.dot(x_ref[...], w_ref[...],
                    preferred_element_type=jnp.float32)
        o_ref[0] = p[:, :m2]
        o_ref[1] = p[:, m2:]

    return pl.pallas_call(
        body,
        grid=(n // ROWBLK,),
        in_specs=[
            pl.BlockSpec((ROWBLK, k), lambda i: (i, 0)),
            pl.BlockSpec((k, m), lambda i: (0, 0)),
        ],
        out_specs=pl.BlockSpec((NC, ROWBLK, m2), lambda i: (0, i, 0)),
        out_shape=jax.ShapeDtypeStruct((NC, n, m2), jnp.float32),
    )(x, wt)


def _tc_post(agg, cnt, h_in, wrt, bl, wlnt):
    n, d_in = h_in.shape
    d = wrt.shape[1]
    d2 = d // 2
    dn = wlnt.shape[1]
    dn2 = dn // 2

    def body(a_ref, c_ref, h_ref, wr_ref, bl_ref, wl_ref, ho_ref, po_ref):
        c = c_ref[:, 0:1]
        inv = 1.0 / jnp.maximum(c, 1.0)
        root = jnp.dot(h_ref[...], wr_ref[...],
                       preferred_element_type=jnp.float32)
        a = jnp.concatenate([a_ref[0], a_ref[1]], axis=1)
        h = jnp.maximum(a * inv + bl_ref[...] + root, 0.0)
        ho_ref[...] = h
        p = jnp.dot(h, wl_ref[...], preferred_element_type=jnp.float32)
        po_ref[0] = p[:, :dn2]
        po_ref[1] = p[:, dn2:]

    return pl.pallas_call(
        body,
        grid=(n // ROWBLK,),
        in_specs=[
            pl.BlockSpec((NC, ROWBLK, d2), lambda i: (0, i, 0)),
            pl.BlockSpec((ROWBLK, CW), lambda i: (i, 0)),
            pl.BlockSpec((ROWBLK, d_in), lambda i: (i, 0)),
            pl.BlockSpec((d_in, d), lambda i: (0, 0)),
            pl.BlockSpec((1, d), lambda i: (0, 0)),
            pl.BlockSpec((d, dn), lambda i: (0, 0)),
        ],
        out_specs=[
            pl.BlockSpec((ROWBLK, d), lambda i: (i, 0)),
            pl.BlockSpec((NC, ROWBLK, dn2), lambda i: (0, i, 0)),
        ],
        out_shape=[
            jax.ShapeDtypeStruct((n, d), jnp.float32),
            jax.ShapeDtypeStruct((NC, n, dn2), jnp.float32),
        ],
    )(agg, cnt, h_in, wrt, bl, wlnt)


def _tc_final(agg, cnt, h_in, wrt, bl, wct, bc):
    n, d_in = h_in.shape
    d = wrt.shape[1]
    d2 = d // 2
    m = wct.shape[1]

    def body(a_ref, c_ref, h_ref, wr_ref, bl_ref, wc_ref, bc_ref, o_ref):
        c = c_ref[:, 0:1]
        inv = 1.0 / jnp.maximum(c, 1.0)
        root = jnp.dot(h_ref[...], wr_ref[...],
                       preferred_element_type=jnp.float32)
        a = jnp.concatenate([a_ref[0], a_ref[1]], axis=1)
        h = jnp.maximum(a * inv + bl_ref[...] + root, 0.0)
        o_ref[...] = jnp.dot(h, wc_ref[...],
                             preferred_element_type=jnp.float32) + bc_ref[...]

    return pl.pallas_call(
        body,
        grid=(n // ROWBLK,),
        in_specs=[
            pl.BlockSpec((NC, ROWBLK, d2), lambda i: (0, i, 0)),
            pl.BlockSpec((ROWBLK, CW), lambda i: (i, 0)),
            pl.BlockSpec((ROWBLK, d_in), lambda i: (i, 0)),
            pl.BlockSpec((d_in, d), lambda i: (0, 0)),
            pl.BlockSpec((1, d), lambda i: (0, 0)),
            pl.BlockSpec((d, m), lambda i: (0, 0)),
            pl.BlockSpec((1, m), lambda i: (0, 0)),
        ],
        out_specs=pl.BlockSpec((ROWBLK, m), lambda i: (i, 0)),
        out_shape=jax.ShapeDtypeStruct((n, m), jnp.float32),
    )(agg, cnt, h_in, wrt, bl, wct, bc)


@jax.jit
def kernel(x, edge_index, Wl1, bl1, Wr1, Wl2, bl2, Wr2, Wl3, bl3, Wr3,
           Wc, bc):
    pad_s = jnp.zeros((EPAD - E,), jnp.int32)
    pad_d = jnp.full((EPAD - E,), DUMP, jnp.int32)
    src2d = jnp.concatenate([edge_index[0], pad_s]).reshape(EROWS, CH)
    dst2d = jnp.concatenate([edge_index[1], pad_d]).reshape(EROWS, CH)

    p1 = _tc_pre(x, Wl1.T)
    agg1, cnt = _sc_pass(p1, src2d, dst2d, True)
    h1, p2 = _tc_post(agg1, cnt, x, Wr1.T, bl1.reshape(1, -1), Wl2.T)
    agg2, = _sc_pass(p2, src2d, dst2d, False)
    h2, p3 = _tc_post(agg2, cnt, h1, Wr2.T, bl2.reshape(1, -1), Wl3.T)
    agg3, = _sc_pass(p3, src2d, dst2d, False)
    return _tc_final(agg3, cnt, h2, Wr3.T, bl3.reshape(1, -1), Wc.T,
                     bc.reshape(1, -1))
